# baseline (speedup 1.0000x reference)
import functools

import jax
import jax.numpy as jnp
from jax import lax
from jax.experimental import pallas as pl
from jax.experimental.pallas import tpu as pltpu
from jax.experimental.pallas import tpu_sc as plsc

HIDDEN = 1024
NUM_ROWS = 2
BATCH = 4
SEQ_LEN = 8192
TOTAL = BATCH * SEQ_LEN

NC = 2
NS = 16
NW = NC * NS

CHUNK = 32
PER_W = TOTAL // NW
NCHUNK = PER_W // CHUNK

_mesh = plsc.VectorSubcoreMesh(core_axis_name="c", subcore_axis_name="s")


@functools.partial(
    pl.kernel,
    mesh=_mesh,
    out_type=jax.ShapeDtypeStruct((TOTAL, HIDDEN), jnp.float32),
    scratch_types=[
        pltpu.VMEM((NCHUNK, CHUNK), jnp.int32),
        pltpu.VMEM((CHUNK, HIDDEN), jnp.float32),
        pltpu.VMEM((CHUNK, HIDDEN), jnp.float32),
        pltpu.SemaphoreType.DMA,
        pltpu.SemaphoreType.DMA,
        pltpu.SemaphoreType.DMA,
        pltpu.SemaphoreType.DMA,
    ],
)
def _sc_lookup(seg_hbm, table_hbm, out_hbm, idx_v, buf0, buf1,
               gs0, gs1, ss0, ss1):
    wid = lax.axis_index("s") * NC + lax.axis_index("c")
    base_chunk = wid * NCHUNK
    pltpu.sync_copy(seg_hbm.at[pl.ds(base_chunk, NCHUNK)], idx_v)

    bufs = (buf0, buf1)
    gsems = (gs0, gs1)
    ssems = (ss0, ss1)
    ga = [None, None]
    sc = [None, None]
    ga[0] = pltpu.async_copy(table_hbm.at[idx_v.at[0]], bufs[0], gsems[0])
    for g in range(NCHUNK):
        b = g & 1
        nb = b ^ 1
        ga[b].wait()
        if g + 1 < NCHUNK:
            if sc[nb] is not None:
                sc[nb].wait()
            ga[nb] = pltpu.async_copy(
                table_hbm.at[idx_v.at[g + 1]], bufs[nb], gsems[nb])
        sc[b] = pltpu.async_copy(
            bufs[b],
            out_hbm.at[pl.ds((base_chunk + g) * CHUNK, CHUNK)],
            ssems[b])
    sc[0].wait()
    sc[1].wait()


def kernel(segments, table):
    seg = segments.astype(jnp.int32).reshape(NW * NCHUNK, CHUNK)
    out = _sc_lookup(seg, table)
    return out.reshape(BATCH, SEQ_LEN, HIDDEN)
